# Initial kernel scaffold; baseline (speedup 1.0000x reference)
import jax
import jax.numpy as jnp
from jax.experimental import pallas as pl


def kernel(x, table, W1, b1, W2, b2):
    raise NotImplementedError("write your pallas kernel here")


# speedup vs baseline: 9.7452x; 9.7452x over previous
import functools

import jax
import jax.numpy as jnp
from jax import lax
from jax.experimental import pallas as pl
from jax.experimental.pallas import tpu as pltpu
from jax.experimental.pallas import tpu_sc as plsc

_D = 32
_H1 = 64
_H2 = 32


def _gather_rows(table, idx_flat):
    BF = idx_flat.shape[0]
    info = plsc.get_sparse_core_info()
    NC, NS = info.num_cores, info.num_subcores
    NW = NC * NS
    per_w = BF // NW
    C = 3328
    assert per_w % C == 0
    n_chunks = per_w // C
    mesh = plsc.VectorSubcoreMesh(core_axis_name="c", subcore_axis_name="s")

    @functools.partial(
        pl.kernel,
        out_type=jax.ShapeDtypeStruct((BF, _D), jnp.float32),
        mesh=mesh,
        scratch_types=[
            pltpu.VMEM((C,), jnp.int32),
            pltpu.VMEM((C, _D), jnp.float32),
            pltpu.SemaphoreType.DMA,
        ],
        compiler_params=pltpu.CompilerParams(use_tc_tiling_on_sc=False),
    )
    def gather_kernel(idx_hbm, table_hbm, out_hbm, idx_v, rows_v, sem):
        wid = lax.axis_index("s") * NC + lax.axis_index("c")
        for i in range(n_chunks):
            base = wid * per_w + i * C
            pltpu.sync_copy(idx_hbm.at[pl.ds(base, C)], idx_v)
            pltpu.async_copy(table_hbm.at[idx_v], rows_v, sem).wait()
            pltpu.sync_copy(rows_v, out_hbm.at[pl.ds(base, C)])

    return gather_kernel(idx_flat, table)


def _mlp(g, W1, b1, W2, b2):
    BF = g.shape[0]
    TM = 8192
    assert BF % TM == 0

    def body(g_ref, w1_ref, b1_ref, w2_ref, b2_ref, o_ref):
        h = jnp.dot(g_ref[...], w1_ref[...], preferred_element_type=jnp.float32)
        h = jnp.maximum(h + b1_ref[...], 0.0)
        o_ref[...] = (
            jnp.dot(h, w2_ref[...], preferred_element_type=jnp.float32)
            + b2_ref[...]
        )

    return pl.pallas_call(
        body,
        grid=(BF // TM,),
        in_specs=[
            pl.BlockSpec((TM, _D), lambda i: (i, 0)),
            pl.BlockSpec((_D, _H1), lambda i: (0, 0)),
            pl.BlockSpec((1, _H1), lambda i: (0, 0)),
            pl.BlockSpec((_H1, _H2), lambda i: (0, 0)),
            pl.BlockSpec((1, _H2), lambda i: (0, 0)),
        ],
        out_specs=pl.BlockSpec((TM, _H2), lambda i: (i, 0)),
        out_shape=jax.ShapeDtypeStruct((BF, _H2), jnp.float32),
    )(g, W1, b1, W2, b2)


def kernel(x, table, W1, b1, W2, b2):
    B, F = x.shape
    idx_flat = x.reshape(-1).astype(jnp.int32)
    g = _gather_rows(table, idx_flat)
    out = _mlp(g, W1, b1.reshape(1, _H1), W2, b2.reshape(1, _H2))
    return out.reshape(B, F, _H2)


# speedup vs baseline: 13.6795x; 1.4037x over previous
import functools

import jax
import jax.numpy as jnp
from jax import lax
from jax.experimental import pallas as pl
from jax.experimental.pallas import tpu as pltpu
from jax.experimental.pallas import tpu_sc as plsc

_D = 32
_H1 = 64
_H2 = 32


def _gather_rows(table, idx_flat):
    BF = idx_flat.shape[0]
    info = plsc.get_sparse_core_info()
    NC, NS = info.num_cores, info.num_subcores
    NW = NC * NS
    per_w = BF // NW
    C = 3328
    assert per_w % C == 0
    n_chunks = per_w // C
    mesh = plsc.VectorSubcoreMesh(core_axis_name="c", subcore_axis_name="s")

    @functools.partial(
        pl.kernel,
        out_type=jax.ShapeDtypeStruct((BF, _D), jnp.float32),
        mesh=mesh,
        scratch_types=[
            pltpu.VMEM((C,), jnp.int32),
            pltpu.VMEM((C, _D), jnp.float32),
            pltpu.SemaphoreType.DMA,
        ],
        compiler_params=pltpu.CompilerParams(use_tc_tiling_on_sc=False),
    )
    def gather_kernel(idx_hbm, table_hbm, out_hbm, idx_v, rows_v, sem):
        wid = lax.axis_index("s") * NC + lax.axis_index("c")
        for i in range(n_chunks):
            base = wid * per_w + i * C
            pltpu.sync_copy(idx_hbm.at[pl.ds(base, C)], idx_v)
            pltpu.async_copy(table_hbm.at[idx_v], rows_v, sem).wait()
            pltpu.sync_copy(rows_v, out_hbm.at[pl.ds(base, C)])

    return gather_kernel(idx_flat, table)


def _mlp_packed(g4, W1d, b1d, W2d, b2d):
    BF4 = g4.shape[0]
    TM = 2048
    assert BF4 % TM == 0

    def body(g_ref, w1_ref, b1_ref, w2_ref, b2_ref, o_ref):
        h = jnp.dot(g_ref[...], w1_ref[...], preferred_element_type=jnp.float32)
        h = jnp.maximum(h + b1_ref[...], 0.0)
        o_ref[...] = (
            jnp.dot(h, w2_ref[...], preferred_element_type=jnp.float32)
            + b2_ref[...]
        )

    return pl.pallas_call(
        body,
        grid=(BF4 // TM,),
        in_specs=[
            pl.BlockSpec((TM, 4 * _D), lambda i: (i, 0)),
            pl.BlockSpec((4 * _D, 4 * _H1), lambda i: (0, 0)),
            pl.BlockSpec((1, 4 * _H1), lambda i: (0, 0)),
            pl.BlockSpec((4 * _H1, 4 * _H2), lambda i: (0, 0)),
            pl.BlockSpec((1, 4 * _H2), lambda i: (0, 0)),
        ],
        out_specs=pl.BlockSpec((TM, 4 * _H2), lambda i: (i, 0)),
        out_shape=jax.ShapeDtypeStruct((BF4, 4 * _H2), jnp.float32),
    )(g4, W1d, b1d, W2d, b2d)


def _block_diag4(W):
    a, b = W.shape
    out = jnp.zeros((4 * a, 4 * b), dtype=W.dtype)
    for r in range(4):
        out = out.at[r * a : (r + 1) * a, r * b : (r + 1) * b].set(W)
    return out


def kernel(x, table, W1, b1, W2, b2):
    B, F = x.shape
    idx_flat = x.T.reshape(-1).astype(jnp.int32)
    g = _gather_rows(table, idx_flat)
    g4 = g.reshape((B * F) // 4, 4 * _D)
    W1d = _block_diag4(W1)
    W2d = _block_diag4(W2)
    b1d = jnp.tile(b1, 4).reshape(1, 4 * _H1)
    b2d = jnp.tile(b2, 4).reshape(1, 4 * _H2)
    o4 = _mlp_packed(g4, W1d, b1d, W2d, b2d)
    out = o4.reshape(F, B, _H2).transpose(1, 0, 2)
    return out
